# Initial kernel scaffold; baseline (speedup 1.0000x reference)
import jax
import jax.numpy as jnp
from jax.experimental import pallas as pl


def kernel(x, edge_index, dist, W1, b1, W2, b2):
    raise NotImplementedError("write your pallas kernel here")


# speedup vs baseline: 6.9154x; 6.9154x over previous
import jax
import jax.numpy as jnp
import numpy as np
from jax import lax
from jax.experimental import pallas as pl
from jax.experimental.pallas import tpu as pltpu
from jax.experimental.pallas import tpu_sc as plsc

N_NODES = 10000
N_EDGES = 320000
IN_FEATS = 128
OUT_FEATS = 128
ONSET = 0.8
CUTOFF = 1.0
GAMMA = float(IN_FEATS) / (CUTOFF - 0.0)
DELTA = CUTOFF / float(IN_FEATS - 1)

NC = 2
NS = 16
NW = NC * NS
FPW = IN_FEATS // NW
XSL = FPW * N_NODES
CHUNK = 4000
NCHUNK = N_EDGES // CHUNK
NGROUP = CHUNK // 16

N_PAD = 10240
BN = 2048

CUT_POLY = (1.0000802954828545, -0.004472879735494428, -2.4076015098281327,
            -0.32647107298962685, 2.8973081193125845, -1.1589232477250402)

LOG2 = float(np.log(2.0))

_MUK = np.linspace(0.0, CUTOFF, IN_FEATS)
_CF = np.exp(-2.0 * GAMMA * DELTA * _MUK - GAMMA * DELTA * DELTA)
_PTAB = np.zeros((NW, 16), dtype=np.float32)
for _w in range(NW):
    _f0 = _w * FPW
    _PTAB[_w, 0] = _MUK[_f0]
    _PTAB[_w, 1:FPW] = _CF[_f0:_f0 + FPW - 1]
_PTAB_FLAT = _PTAB.reshape(-1)


def _sc_body(ptab_hbm, xt_hbm, src_hbm, dst_hbm, dist_hbm, out_hbm,
             xv, hv, sv0, sv1, dv0, dv1, rv0, rv1, ps, sems):
    svs, dvs, rvs = (sv0, sv1), (dv0, dv1), (rv0, rv1)
    wid = lax.axis_index("s") * NC + lax.axis_index("c")

    pltpu.sync_copy(ptab_hbm.at[pl.ds(wid * 16, 16)], ps)
    pltpu.sync_copy(xt_hbm.at[pl.ds(wid * XSL, XSL)], xv)

    zeros16 = jnp.zeros((16,), jnp.float32)

    @plsc.parallel_loop(0, XSL // 16, unroll=8)
    def _zero(i):
        hv[pl.ds(i * 16, 16)] = zeros16

    pvec = ps[...]
    m0 = pvec[0]
    qscale = jnp.float32(2.0 * GAMMA * DELTA)
    cks = [pvec[1 + j] for j in range(FPW - 1)]

    def start_chunk(b, chunk_base):
        pltpu.async_copy(src_hbm.at[pl.ds(chunk_base, CHUNK)], svs[b], sems.at[b])
        pltpu.async_copy(dst_hbm.at[pl.ds(chunk_base, CHUNK)], dvs[b], sems.at[b])
        pltpu.async_copy(dist_hbm.at[pl.ds(chunk_base, CHUNK)], rvs[b], sems.at[b])

    def wait_chunk(b):
        pltpu.make_async_copy(src_hbm.at[pl.ds(0, CHUNK)], svs[b], sems.at[b]).wait()
        pltpu.make_async_copy(dst_hbm.at[pl.ds(0, CHUNK)], dvs[b], sems.at[b]).wait()
        pltpu.make_async_copy(dist_hbm.at[pl.ds(0, CHUNK)], rvs[b], sems.at[b]).wait()

    for b in range(2):
        start_chunk(b, b * CHUNK)

    def chunk_pair(cc, carry):
        for b in range(2):
            chunk = cc * 2 + b
            wait_chunk(b)

            @plsc.parallel_loop(0, NGROUP, unroll=4)
            def _group(g):
                s16 = svs[b][pl.ds(g * 16, 16)]
                d16 = dvs[b][pl.ds(g * 16, 16)]
                r = rvs[b][pl.ds(g * 16, 16)]

                t = jnp.clip((r - ONSET) * (1.0 / (CUTOFF - ONSET)), 0.0, 1.0)
                cut = jnp.float32(CUT_POLY[-1])
                for coef in CUT_POLY[-2::-1]:
                    cut = cut * t + jnp.float32(coef)

                a = r - m0
                e = jnp.exp(a * a * (-GAMMA)) * cut
                q = jnp.exp(r * qscale)

                for f in range(FPW):
                    off = f * N_NODES
                    xg = plsc.load_gather(xv, [s16 + off])
                    plsc.addupdate_scatter(hv, [d16 + off], xg * e)
                    if f < FPW - 1:
                        e = e * (q * cks[f])

            @pl.when(chunk + 2 < NCHUNK)
            def _prefetch():
                start_chunk(b, (chunk + 2) * CHUNK)

        return carry

    lax.fori_loop(0, NCHUNK // 2, chunk_pair, 0)

    pltpu.sync_copy(hv, out_hbm.at[pl.ds(wid * XSL, XSL)])


def _sc_scatter(ptab, xt_flat, src, dst, dist):
    mesh = plsc.VectorSubcoreMesh(core_axis_name="c", subcore_axis_name="s")
    fn = pl.kernel(
        _sc_body,
        out_type=jax.ShapeDtypeStruct((IN_FEATS * N_NODES,), jnp.float32),
        mesh=mesh,
        compiler_params=pltpu.CompilerParams(needs_layout_passes=False),
        scratch_types=[
            pltpu.VMEM((XSL,), jnp.float32),
            pltpu.VMEM((XSL,), jnp.float32),
            pltpu.VMEM((CHUNK,), jnp.int32),
            pltpu.VMEM((CHUNK,), jnp.int32),
            pltpu.VMEM((CHUNK,), jnp.int32),
            pltpu.VMEM((CHUNK,), jnp.int32),
            pltpu.VMEM((CHUNK,), jnp.float32),
            pltpu.VMEM((CHUNK,), jnp.float32),
            pltpu.VMEM((16,), jnp.float32),
            pltpu.SemaphoreType.DMA((2,)),
        ],
    )
    return fn(ptab, xt_flat, src, dst, dist)


def _mlp_body(ht_ref, w1_ref, b1_ref, w2_ref, b2_ref, o_ref):
    ht = ht_ref[...]
    z = lax.dot_general(ht, w1_ref[...], (((0,), (0,)), ((), ())),
                        preferred_element_type=jnp.float32)
    z = z + b1_ref[...]
    sp = jnp.maximum(z, 0.0) + jnp.log(1.0 + jnp.exp(-jnp.abs(z))) - LOG2
    o_ref[...] = lax.dot_general(sp, w2_ref[...], (((1,), (0,)), ((), ())),
                                 preferred_element_type=jnp.float32) + b2_ref[...]


def _mlp(ht_pad, W1, b1, W2, b2):
    grid = (N_PAD // BN,)
    return pl.pallas_call(
        _mlp_body,
        grid=grid,
        in_specs=[
            pl.BlockSpec((IN_FEATS, BN), lambda i: (0, i)),
            pl.BlockSpec((IN_FEATS, OUT_FEATS), lambda i: (0, 0)),
            pl.BlockSpec((1, OUT_FEATS), lambda i: (0, 0)),
            pl.BlockSpec((OUT_FEATS, OUT_FEATS), lambda i: (0, 0)),
            pl.BlockSpec((1, OUT_FEATS), lambda i: (0, 0)),
        ],
        out_specs=pl.BlockSpec((BN, OUT_FEATS), lambda i: (i, 0)),
        out_shape=jax.ShapeDtypeStruct((N_PAD, OUT_FEATS), jnp.float32),
    )(ht_pad, W1, b1, W2, b2)


def kernel(x, edge_index, dist, W1, b1, W2, b2):
    src = edge_index[0]
    dst = edge_index[1]
    xt_flat = x.T.reshape(-1)
    ptab = jnp.asarray(_PTAB_FLAT)
    h_flat = _sc_scatter(ptab, xt_flat, src, dst, dist)
    ht = h_flat.reshape(IN_FEATS, N_NODES)
    ht_pad = jnp.pad(ht, ((0, 0), (0, N_PAD - N_NODES)))
    out_pad = _mlp(ht_pad, W1, b1.reshape(1, -1), W2, b2.reshape(1, -1))
    return out_pad[:N_NODES]
